# Initial kernel scaffold; baseline (speedup 1.0000x reference)
import jax
import jax.numpy as jnp
from jax.experimental import pallas as pl


def kernel(x, edge_index, batch, edge_weights, W1, b1, W2, b2, l1W, l1b, l3W, l3b, l4W, l4b):
    raise NotImplementedError("write your pallas kernel here")


# speedup vs baseline: 2.2364x; 2.2364x over previous
import jax
import jax.numpy as jnp
from jax.experimental import pallas as pl

_SLOPE = (1.0 / 8.0 + 1.0 / 3.0) / 2.0
_NB = 1000


def _rrelu(v):
    return jnp.where(v >= 0, v, v * _SLOPE)


def _mm_body(x_ref, w_ref, o_ref):
    o_ref[...] = jnp.dot(x_ref[...], w_ref[...],
                         preferred_element_type=jnp.float32)


def _node_matmul(x, w):
    n, f = x.shape
    h = w.shape[1]
    grid = (n // _NB,)
    return pl.pallas_call(
        _mm_body,
        grid=grid,
        in_specs=[
            pl.BlockSpec((_NB, f), lambda i: (i, 0)),
            pl.BlockSpec((f, h), lambda i: (0, 0)),
        ],
        out_specs=pl.BlockSpec((_NB, h), lambda i: (i, 0)),
        out_shape=jax.ShapeDtypeStruct((n, h), jnp.float32),
    )(x, w)


def _post_body(acc_ref, hsrc_ref, dis_ref, b_ref, o_ref):
    dis = dis_ref[...]
    o_ref[...] = dis * acc_ref[...] + (dis * dis) * hsrc_ref[...] + b_ref[...]


def _gcn_epilogue(acc, h_self, dis, b):
    n, h = acc.shape
    grid = (n // _NB,)
    return pl.pallas_call(
        _post_body,
        grid=grid,
        in_specs=[
            pl.BlockSpec((_NB, h), lambda i: (i, 0)),
            pl.BlockSpec((_NB, h), lambda i: (i, 0)),
            pl.BlockSpec((_NB, 1), lambda i: (i, 0)),
            pl.BlockSpec((1, h), lambda i: (0, 0)),
        ],
        out_specs=pl.BlockSpec((_NB, h), lambda i: (i, 0)),
        out_shape=jax.ShapeDtypeStruct((n, h), jnp.float32),
    )(acc, h_self, dis, b)


def _resrelu_body(a_ref, b_ref, o_ref):
    o_ref[...] = jnp.maximum(a_ref[...] + b_ref[...], 0.0)


def _res_relu(a, b):
    n, h = a.shape
    grid = (n // _NB,)
    return pl.pallas_call(
        _resrelu_body,
        grid=grid,
        in_specs=[
            pl.BlockSpec((_NB, h), lambda i: (i, 0)),
            pl.BlockSpec((_NB, h), lambda i: (i, 0)),
        ],
        out_specs=pl.BlockSpec((_NB, h), lambda i: (i, 0)),
        out_shape=jax.ShapeDtypeStruct((n, h), jnp.float32),
    )(a, b)


def _mlp_body(g_ref, w1_ref, b1_ref, w3_ref, b3_ref, w4_ref, b4_ref, o_ref):
    g = g_ref[...]
    g = _rrelu(g + jnp.dot(g, w1_ref[...], preferred_element_type=jnp.float32)
               + b1_ref[...])
    g = _rrelu(g + jnp.dot(g, w3_ref[...], preferred_element_type=jnp.float32)
               + b3_ref[...])
    g = _rrelu(jnp.dot(g, w4_ref[...], preferred_element_type=jnp.float32)
               + b4_ref[...])
    o_ref[...] = g


def _final_mlp(g, l1W, l1b, l3W, l3b, l4W, l4b):
    bsz, h = g.shape
    return pl.pallas_call(
        _mlp_body,
        in_specs=[
            pl.BlockSpec((bsz, h), lambda: (0, 0)),
            pl.BlockSpec((h, h), lambda: (0, 0)),
            pl.BlockSpec((1, h), lambda: (0, 0)),
            pl.BlockSpec((h, h), lambda: (0, 0)),
            pl.BlockSpec((1, h), lambda: (0, 0)),
            pl.BlockSpec((h, 1), lambda: (0, 0)),
            pl.BlockSpec((1, 1), lambda: (0, 0)),
        ],
        out_specs=pl.BlockSpec((bsz, 1), lambda: (0, 0)),
        out_shape=jax.ShapeDtypeStruct((bsz, 1), jnp.float32),
    )(g, l1W, l1b.reshape(1, h), l3W, l3b.reshape(1, h),
      l4W, l4b.reshape(1, 1))


def kernel(x, edge_index, batch, edge_weights, W1, b1, W2, b2,
           l1W, l1b, l3W, l3b, l4W, l4b):
    n = x.shape[0]
    src = edge_index[0]
    dst = edge_index[1]

    deg1 = jax.ops.segment_sum(edge_weights, dst, num_segments=n) + 1.0
    dis1 = jax.lax.rsqrt(deg1)[:, None]
    ones_e = jnp.ones(src.shape, jnp.float32)
    deg2 = jax.ops.segment_sum(ones_e, dst, num_segments=n) + 1.0
    dis2 = jax.lax.rsqrt(deg2)[:, None]

    h1 = _node_matmul(x, W1)
    g1 = dis1 * h1
    acc1 = jax.ops.segment_sum(edge_weights[:, None] * g1[src], dst,
                               num_segments=n)
    out1 = _gcn_epilogue(acc1, h1, dis1, b1.reshape(1, -1))

    hp = jnp.maximum(out1, jax.ops.segment_max(out1[src], dst,
                                               num_segments=n))

    h2 = _node_matmul(hp, W2)
    g2 = dis2 * h2
    acc2 = jax.ops.segment_sum(g2[src], dst, num_segments=n)
    out2 = _gcn_epilogue(acc2, h2, dis2, b2.reshape(1, -1))

    hh = _res_relu(hp, out2)

    g = jax.ops.segment_max(hh, batch, num_segments=64)

    return _final_mlp(g, l1W, l1b, l3W, l3b, l4W, l4b)
